# baseline (speedup 1.0000x reference)
import functools

import jax
import jax.numpy as jnp
from jax import lax
from jax.experimental import pallas as pl
from jax.experimental.pallas import tpu as pltpu
from jax.experimental.pallas import tpu_sc as plsc

_NT = 8
_TW = 32
_CR = 128
_L = 16


@functools.lru_cache(maxsize=None)
def _build(total, nseg):
    n_chunks = total // _CR
    assert n_chunks * _CR == total
    info = plsc.get_sparse_core_info()
    nc, ns = info.num_cores, info.num_subcores
    nw = nc * ns
    ch_per_w = -(-n_chunks // nw)
    assert ch_per_w % 2 == 0
    npad = (-(-(nseg + 1) // _L) + 1) * _L
    w0 = 1 << (nseg.bit_length() - 2)

    mesh = plsc.VectorSubcoreMesh(core_axis_name="c", subcore_axis_name="s")

    @functools.partial(
        pl.kernel,
        mesh=mesh,
        compiler_params=pltpu.CompilerParams(
            needs_layout_passes=False, use_tc_tiling_on_sc=False
        ),
        out_type=jax.ShapeDtypeStruct((n_chunks, _NT, _CR, _TW), jnp.float32),
        scratch_types=[
            pltpu.VMEM((npad,), jnp.int32),
            pltpu.VMEM((2, _NT, _CR), jnp.int32),
            pltpu.VMEM((2, _NT, _CR, _TW), jnp.float32),
            pltpu.SemaphoreType.DMA,
            pltpu.SemaphoreType.DMA,
            pltpu.SemaphoreType.DMA,
            pltpu.SemaphoreType.DMA,
        ],
    )
    def _k(flat_hbm, starts_hbm, out_hbm, starts_v, idx_v, buf_v,
           sg0, sg1, ss0, ss1):
        wid = lax.axis_index("s") * nc + lax.axis_index("c")

        pltpu.sync_copy(starts_hbm, starts_v)

        lane = lax.iota(jnp.int32, _L)
        pos_base = ((lane >> 3) << 6) + ((lane & 7) << 2)

        def compute_idx(p, c):
            base = c * _CR
            pv = jnp.full((_L,), p, jnp.int32)

            def vbody(v, carry):
                rowv = base + v * _L + lane
                jv = jnp.full((_L,), v, jnp.int32)
                lo = jnp.zeros((_L,), jnp.int32)
                w = w0
                while w >= 1:
                    cand = lo + w
                    sv = plsc.load_gather(starts_v, [cand])
                    lo = jnp.where(sv <= rowv, cand, lo)
                    w >>= 1
                sstart = plsc.load_gather(starts_v, [lo])
                slen = plsc.load_gather(starts_v, [lo + 1]) - sstart
                local = rowv - sstart
                for i in range(_NT):
                    if i == 0:
                        src = rowv
                    else:
                        sh = 1 << (i - 1)
                        src = sstart + lax.rem(local + sh, slen)
                    const_i = ((i >> 2) << 5) + (i & 3)
                    g = ((src >> 3) << 6) + ((src & 7) << 2) + const_i
                    plsc.store_scatter(idx_v, [pv, jv, pos_base + const_i], g)
                return carry

            lax.fori_loop(0, _CR // _L, vbody, jnp.int32(0))

        def fire_gathers(p, sem):
            for j in range(_NT):
                pltpu.async_copy(flat_hbm.at[idx_v.at[p, j]], buf_v.at[p, j],
                                 sem)

        def drain(p, sem):
            pltpu.make_async_copy(out_hbm.at[0], buf_v.at[p], sem).wait()

        def chunk_id(k):
            return jnp.minimum(wid * ch_per_w + k, n_chunks - 1)

        def pair_body(kk, carry):
            c_a = chunk_id(2 * kk)
            c_b = chunk_id(2 * kk + 1)
            not_first = kk > 0

            @pl.when(not_first)
            def _():
                drain(0, ss0)

            compute_idx(0, c_a)
            fire_gathers(0, sg0)

            @pl.when(not_first)
            def _():
                drain(1, sg1)
                pltpu.async_copy(buf_v.at[1], out_hbm.at[chunk_id(2 * kk - 1)],
                                 ss1)

            compute_idx(1, c_b)

            @pl.when(not_first)
            def _():
                drain(1, ss1)

            fire_gathers(1, sg1)
            drain(0, sg0)
            pltpu.async_copy(buf_v.at[0], out_hbm.at[c_a], ss0)
            return carry

        lax.fori_loop(0, ch_per_w // 2, pair_body, jnp.int32(0))

        drain(1, sg1)
        last_b = chunk_id(ch_per_w - 1)
        scp = pltpu.async_copy(buf_v.at[1], out_hbm.at[last_b], ss1)
        drain(0, ss0)
        scp.wait()

    return _k


def kernel(data, lengths):
    total, emb = data.shape
    assert emb == _NT * _TW
    nseg = lengths.shape[0]
    npad = (-(-(nseg + 1) // _L) + 1) * _L
    lens = lengths.astype(jnp.int32)
    flat = (data.reshape(total // 8, 8, 2, 4, _TW)
            .transpose(0, 2, 1, 3, 4)
            .reshape(total * _NT, _TW))
    starts = jnp.concatenate([
        jnp.zeros((1,), jnp.int32),
        jnp.cumsum(lens),
        jnp.full((npad - nseg - 1,), jnp.int32(total)),
    ])
    out = _build(total, nseg)(flat, starts)
    return (out.reshape(total // 8, 2, 8, 4, _TW)
            .transpose(0, 2, 1, 3, 4)
            .reshape(total, emb))


# speedup vs baseline: 4.4395x; 4.4395x over previous
import functools

import jax
import jax.numpy as jnp
from jax import lax
from jax.experimental import pallas as pl
from jax.experimental.pallas import tpu as pltpu
from jax.experimental.pallas import tpu_sc as plsc

_NT = 8
_TW = 32
_CR = 128
_L = 16


@functools.lru_cache(maxsize=None)
def _build(total, nseg):
    n_chunks = total // _CR
    assert n_chunks * _CR == total
    info = plsc.get_sparse_core_info()
    nc, ns = info.num_cores, info.num_subcores
    nw = nc * ns
    ch_per_w = -(-n_chunks // nw)
    assert ch_per_w % 2 == 0
    npad = (-(-(nseg + 1) // _L) + 1) * _L
    w0 = 1 << (nseg.bit_length() - 2)

    mesh = plsc.VectorSubcoreMesh(core_axis_name="c", subcore_axis_name="s")

    @functools.partial(
        pl.kernel,
        mesh=mesh,
        compiler_params=pltpu.CompilerParams(
            needs_layout_passes=False, use_tc_tiling_on_sc=False
        ),
        out_type=jax.ShapeDtypeStruct((n_chunks, _NT * _CR, _TW), jnp.float32),
        scratch_types=[
            pltpu.VMEM((npad,), jnp.int32),
            pltpu.VMEM((2, _NT * _CR), jnp.int32),
            pltpu.VMEM((2, _NT * _CR, _TW), jnp.float32),
            pltpu.SemaphoreType.DMA,
            pltpu.SemaphoreType.DMA,
            pltpu.SemaphoreType.DMA,
            pltpu.SemaphoreType.DMA,
        ],
    )
    def _k(flat_hbm, starts_hbm, out_hbm, starts_v, idx_v, buf_v,
           sg0, sg1, ss0, ss1):
        wid = lax.axis_index("s") * nc + lax.axis_index("c")

        pltpu.sync_copy(starts_hbm, starts_v)

        lane = lax.iota(jnp.int32, _L)
        pos_base = lane * _NT

        def compute_idx(p, c):
            base = c * _CR
            pv = jnp.full((_L,), p, jnp.int32)

            def vbody(v, carry):
                rowv = base + v * _L + lane
                vpos = v * (_L * _NT) + pos_base
                lo = jnp.zeros((_L,), jnp.int32)
                w = w0
                while w >= 1:
                    cand = lo + w
                    sv = plsc.load_gather(starts_v, [cand])
                    lo = jnp.where(sv <= rowv, cand, lo)
                    w >>= 1
                sstart = plsc.load_gather(starts_v, [lo])
                slen = plsc.load_gather(starts_v, [lo + 1]) - sstart
                local = rowv - sstart
                for i in range(_NT):
                    if i == 0:
                        g = rowv * _NT
                    else:
                        sh = 1 << (i - 1)
                        g = (sstart + lax.rem(local + sh, slen)) * _NT + i
                    plsc.store_scatter(idx_v, [pv, vpos + i], g)
                return carry

            lax.fori_loop(0, _CR // _L, vbody, jnp.int32(0))

        def fire_gathers(p, sem):
            pltpu.async_copy(flat_hbm.at[idx_v.at[p]], buf_v.at[p], sem)

        def drain(p, sem):
            pltpu.make_async_copy(out_hbm.at[0], buf_v.at[p], sem).wait()

        def chunk_id(k):
            return jnp.minimum(wid * ch_per_w + k, n_chunks - 1)

        def pair_body(kk, carry):
            c_a = chunk_id(2 * kk)
            c_b = chunk_id(2 * kk + 1)
            not_first = kk > 0

            @pl.when(not_first)
            def _():
                drain(0, ss0)

            compute_idx(0, c_a)
            fire_gathers(0, sg0)

            @pl.when(not_first)
            def _():
                drain(1, sg1)
                pltpu.async_copy(buf_v.at[1], out_hbm.at[chunk_id(2 * kk - 1)],
                                 ss1)

            compute_idx(1, c_b)

            @pl.when(not_first)
            def _():
                drain(1, ss1)

            fire_gathers(1, sg1)
            drain(0, sg0)
            pltpu.async_copy(buf_v.at[0], out_hbm.at[c_a], ss0)
            return carry

        lax.fori_loop(0, ch_per_w // 2, pair_body, jnp.int32(0))

        drain(1, sg1)
        last_b = chunk_id(ch_per_w - 1)
        scp = pltpu.async_copy(buf_v.at[1], out_hbm.at[last_b], ss1)
        drain(0, ss0)
        scp.wait()

    return _k


def kernel(data, lengths):
    total, emb = data.shape
    assert emb == _NT * _TW
    nseg = lengths.shape[0]
    npad = (-(-(nseg + 1) // _L) + 1) * _L
    flat = data.reshape(total * _NT, _TW)
    lens = lengths.astype(jnp.int32)
    starts = jnp.concatenate([
        jnp.zeros((1,), jnp.int32),
        jnp.cumsum(lens),
        jnp.full((npad - nseg - 1,), jnp.int32(total)),
    ])
    out = _build(total, nseg)(flat, starts)
    return out.reshape(total, emb)


# speedup vs baseline: 9.4429x; 2.1270x over previous
import functools

import jax
import jax.numpy as jnp
from jax import lax
from jax.experimental import pallas as pl
from jax.experimental.pallas import tpu as pltpu
from jax.experimental.pallas import tpu_sc as plsc

_NT = 8
_TW = 32
_CR = 128
_L = 16


@functools.lru_cache(maxsize=None)
def _build(total, nseg):
    n_chunks = total // _CR
    assert n_chunks * _CR == total
    info = plsc.get_sparse_core_info()
    nc, ns = info.num_cores, info.num_subcores
    nw = nc * ns
    ch_per_w = -(-n_chunks // nw)
    assert ch_per_w % 2 == 0
    npad = (-(-(nseg + 1) // _L) + 1) * _L
    w0 = 1 << (nseg.bit_length() - 2)

    mesh = plsc.VectorSubcoreMesh(core_axis_name="c", subcore_axis_name="s")

    @functools.partial(
        pl.kernel,
        mesh=mesh,
        compiler_params=pltpu.CompilerParams(
            needs_layout_passes=False, use_tc_tiling_on_sc=False
        ),
        out_type=jax.ShapeDtypeStruct((n_chunks, _NT * _CR, _TW), jnp.float32),
        scratch_types=[
            pltpu.VMEM((npad,), jnp.int32),
            pltpu.VMEM((2, _NT * _CR), jnp.int32),
            pltpu.VMEM((2, _NT * _CR, _TW), jnp.float32),
            pltpu.SemaphoreType.DMA,
            pltpu.SemaphoreType.DMA,
            pltpu.SemaphoreType.DMA,
            pltpu.SemaphoreType.DMA,
        ],
    )
    def _k(flat_hbm, starts_hbm, out_hbm, starts_v, idx_v, buf_v,
           sg0, sg1, ss0, ss1):
        wid = lax.axis_index("s") * nc + lax.axis_index("c")

        pltpu.sync_copy(starts_hbm, starts_v)

        lane = lax.iota(jnp.int32, _L)
        pos_base = ((lane >> 3) << 6) + ((lane & 7) << 2)

        def compute_idx(p, c):
            base = c * _CR
            pv = jnp.full((_L,), p, jnp.int32)

            def vbody(v, carry):
                rowv = base + v * _L + lane
                vpos = v * (_L * _NT) + pos_base
                lo = jnp.zeros((_L,), jnp.int32)
                w = w0
                while w >= 1:
                    cand = lo + w
                    sv = plsc.load_gather(starts_v, [cand])
                    lo = jnp.where(sv <= rowv, cand, lo)
                    w >>= 1
                sstart = plsc.load_gather(starts_v, [lo])
                slen = plsc.load_gather(starts_v, [lo + 1]) - sstart
                local = rowv - sstart
                for i in range(_NT):
                    if i == 0:
                        src = rowv
                    else:
                        sh = 1 << (i - 1)
                        src = sstart + lax.rem(local + sh, slen)
                    const_i = ((i >> 2) << 5) + (i & 3)
                    g = ((src >> 3) << 6) + ((src & 7) << 2) + const_i
                    plsc.store_scatter(idx_v, [pv, vpos + const_i], g)
                return carry

            lax.fori_loop(0, _CR // _L, vbody, jnp.int32(0))

        def fire_gathers(p, sem):
            pltpu.async_copy(flat_hbm.at[idx_v.at[p]], buf_v.at[p], sem)

        def drain(p, sem):
            pltpu.make_async_copy(out_hbm.at[0], buf_v.at[p], sem).wait()

        def chunk_id(k):
            return jnp.minimum(wid * ch_per_w + k, n_chunks - 1)

        def pair_body(kk, carry):
            c_a = chunk_id(2 * kk)
            c_b = chunk_id(2 * kk + 1)
            not_first = kk > 0

            @pl.when(not_first)
            def _():
                drain(0, ss0)

            compute_idx(0, c_a)
            fire_gathers(0, sg0)

            @pl.when(not_first)
            def _():
                drain(1, sg1)
                pltpu.async_copy(buf_v.at[1], out_hbm.at[chunk_id(2 * kk - 1)],
                                 ss1)

            compute_idx(1, c_b)

            @pl.when(not_first)
            def _():
                drain(1, ss1)

            fire_gathers(1, sg1)
            drain(0, sg0)
            pltpu.async_copy(buf_v.at[0], out_hbm.at[c_a], ss0)
            return carry

        lax.fori_loop(0, ch_per_w // 2, pair_body, jnp.int32(0))

        drain(1, sg1)
        last_b = chunk_id(ch_per_w - 1)
        scp = pltpu.async_copy(buf_v.at[1], out_hbm.at[last_b], ss1)
        drain(0, ss0)
        scp.wait()

    return _k


def kernel(data, lengths):
    total, emb = data.shape
    assert emb == _NT * _TW
    nseg = lengths.shape[0]
    npad = (-(-(nseg + 1) // _L) + 1) * _L
    flat = (data.reshape(total // 8, 8, 2, 128)
            .transpose(0, 2, 1, 3)
            .reshape(total * _NT, _TW))
    lens = lengths.astype(jnp.int32)
    starts = jnp.concatenate([
        jnp.zeros((1,), jnp.int32),
        jnp.cumsum(lens),
        jnp.full((npad - nseg - 1,), jnp.int32(total)),
    ])
    out = _build(total, nseg)(flat, starts)
    return (out.reshape(total // 8, 2, 8, 128)
            .transpose(0, 2, 1, 3)
            .reshape(total, emb))
